# baseline (speedup 1.0000x reference)
import functools

import jax
import jax.numpy as jnp
import numpy as np
from jax import lax
from jax.experimental import pallas as pl
from jax.experimental.pallas import tpu as pltpu
from jax.experimental.pallas import tpu_sc as plsc

_H = 64
_L = _H * _H
_B = 16
_C = 96
_NW = 32
_NCHUNK_TOTAL = 128
_CHUNKS_PER_W = _NCHUNK_TOTAL // _NW
_RSLAB = 512


def _row_table() -> np.ndarray:
    q = np.arange(512)
    k = q >> 7
    low = q & 127
    il = ((low >> 1) & 1) | (((low >> 3) & 1) << 1) | (((low >> 5) & 1) << 2)
    j = (
        (low & 1)
        | (((low >> 2) & 1) << 1)
        | (((low >> 4) & 1) << 2)
        | (((low >> 6) & 1) << 3)
        | ((k & 1) << 4)
        | ((k >> 1) << 5)
    )
    return (il * 64 + j).astype(np.int32)


def _body(x_hbm, tab_hbm, out_hbm, tab_v, in_v, out_v):
    nc = 2
    wid = lax.axis_index("s") * nc + lax.axis_index("c")
    pltpu.sync_copy(tab_hbm, tab_v)

    def chunk_body(t, _):
        cid = wid * _CHUNKS_PER_W + t
        pltpu.sync_copy(
            x_hbm.at[pl.ds(pl.multiple_of(cid * _RSLAB, _RSLAB), _RSLAB)], in_v
        )

        rows = [tab_v[pl.ds(g * 16, 16)] for g in range(32)]

        @plsc.parallel_loop(0, _C, step=1, unroll=2)
        def c_body(c):
            csplat = jnp.full((16,), c, dtype=jnp.int32)
            for g in range(32):
                vals = plsc.load_gather(in_v, [rows[g], csplat])
                out_v[c, pl.ds(g * 16, 16)] = vals

        b = cid >> 3
        h = cid & 7
        h1 = h & 1
        h2 = (h >> 1) & 1
        h4 = (h >> 2) & 1
        rowbase = pl.multiple_of(b * _C, _C)
        for k in range(4):
            colstart = h1 * 128 + (k & 1) * 256 + h2 * 512 + (k >> 1) * 1024 + h4 * 2048
            pltpu.sync_copy(
                out_v.at[:, pl.ds(k * 128, 128)],
                out_hbm.at[pl.ds(rowbase, _C), pl.ds(pl.multiple_of(colstart, 128), 128)],
            )
        return 0

    lax.fori_loop(0, _CHUNKS_PER_W, chunk_body, 0, unroll=False)


@functools.partial(jax.jit, static_argnames=())
def kernel(x):
    B, C, H, _ = x.shape
    xv = x.transpose(0, 2, 3, 1).reshape(B * H * H, C)
    tab = jnp.asarray(_row_table())
    run = pl.kernel(
        _body,
        out_type=jax.ShapeDtypeStruct((B * C, H * H), jnp.float32),
        mesh=plsc.VectorSubcoreMesh(core_axis_name="c", subcore_axis_name="s"),
        compiler_params=pltpu.CompilerParams(needs_layout_passes=False),
        scratch_types=[
            pltpu.VMEM((_RSLAB,), jnp.int32),
            pltpu.VMEM((_RSLAB, _C), jnp.float32),
            pltpu.VMEM((_C, _RSLAB), jnp.float32),
        ],
    )
    out = run(xv, tab)
    return out.reshape(B, C, H * H)


# speedup vs baseline: 1.1787x; 1.1787x over previous
import functools

import jax
import jax.numpy as jnp
import numpy as np
from jax import lax
from jax.experimental import pallas as pl
from jax.experimental.pallas import tpu as pltpu
from jax.experimental.pallas import tpu_sc as plsc

_H = 64
_L = _H * _H
_B = 16
_C = 96
_NW = 32
_NSUB_TOTAL = 512
_SUBS_PER_W = _NSUB_TOTAL // _NW
_RSUB = 128


def _row_table() -> np.ndarray:
    q = np.arange(_RSUB)
    il = ((q >> 1) & 1) | (((q >> 3) & 1) << 1) | (((q >> 5) & 1) << 2)
    jl = (q & 1) | (((q >> 2) & 1) << 1) | (((q >> 4) & 1) << 2) | (((q >> 6) & 1) << 3)
    return (il * 16 + jl).astype(np.int32)


def _body(x_hbm, tab_hbm, out_hbm, tab_v, in_a, in_b, out_a, out_b,
          in_sem_a, in_sem_b, out_sem_a, out_sem_b):
    nc = 2
    wid = lax.axis_index("s") * nc + lax.axis_index("c")
    pltpu.sync_copy(tab_hbm, tab_v)
    sid0 = wid * _SUBS_PER_W

    def start_in(sid, in_ref, sem):
        cid = sid >> 2
        k = sid & 3
        base = cid * 512 + (k & 1) * 16 + ((k >> 1) & 1) * 32
        for il in range(8):
            pltpu.make_async_copy(
                x_hbm.at[pl.ds(pl.multiple_of(base + il * 64, 16), 16)],
                in_ref.at[pl.ds(il * 16, 16)],
                sem,
            ).start()

    def wait_in(in_ref, sem):
        for il in range(8):
            pltpu.make_async_copy(
                x_hbm.at[pl.ds(0, 16)],
                in_ref.at[pl.ds(il * 16, 16)],
                sem,
            ).wait()

    def out_slices(sid, out_ref):
        cid = sid >> 2
        k = sid & 3
        b = cid >> 3
        h = cid & 7
        colstart = (
            (h & 1) * 128 + (k & 1) * 256 + ((h >> 1) & 1) * 512
            + ((k >> 1) & 1) * 1024 + ((h >> 2) & 1) * 2048
        )
        dst = out_hbm.at[
            pl.ds(pl.multiple_of(b * _C, _C), _C),
            pl.ds(pl.multiple_of(colstart, 128), 128),
        ]
        return out_ref, dst

    def start_out(sid, out_ref, sem):
        src, dst = out_slices(sid, out_ref)
        pltpu.make_async_copy(src, dst, sem).start()

    def wait_out(out_ref, sem):
        pltpu.make_async_copy(
            out_ref, out_hbm.at[pl.ds(0, _C), pl.ds(0, 128)], sem
        ).wait()

    def compute(in_ref, out_ref):
        for g in range(8):
            rows_g = tab_v[pl.ds(g * 16, 16)]

            @plsc.parallel_loop(0, _C, step=1, unroll=4)
            def c_body(c, rows_g=rows_g, g=g):
                csplat = jnp.full((16,), c, dtype=jnp.int32)
                out_ref[c, pl.ds(g * 16, 16)] = plsc.load_gather(
                    in_ref, [rows_g, csplat]
                )

    start_in(sid0, in_a, in_sem_a)
    start_in(sid0 + 1, in_b, in_sem_b)

    def pair_body(t, _):
        u = t * 2

        wait_in(in_a, in_sem_a)

        @pl.when(t > 0)
        def _():
            wait_out(out_a, out_sem_a)

        compute(in_a, out_a)
        start_out(sid0 + u, out_a, out_sem_a)

        @pl.when(t < 7)
        def _():
            start_in(sid0 + u + 2, in_a, in_sem_a)

        wait_in(in_b, in_sem_b)

        @pl.when(t > 0)
        def _():
            wait_out(out_b, out_sem_b)

        compute(in_b, out_b)
        start_out(sid0 + u + 1, out_b, out_sem_b)

        @pl.when(t < 7)
        def _():
            start_in(sid0 + u + 3, in_b, in_sem_b)

        return 0

    lax.fori_loop(0, _SUBS_PER_W // 2, pair_body, 0, unroll=False)
    wait_out(out_a, out_sem_a)
    wait_out(out_b, out_sem_b)


@functools.partial(jax.jit, static_argnames=())
def kernel(x):
    B, C, H, _ = x.shape
    xv = x.transpose(0, 2, 3, 1).reshape(B * H * H, C)
    tab = jnp.asarray(_row_table())
    run = pl.kernel(
        _body,
        out_type=jax.ShapeDtypeStruct((B * C, H * H), jnp.float32),
        mesh=plsc.VectorSubcoreMesh(core_axis_name="c", subcore_axis_name="s"),
        compiler_params=pltpu.CompilerParams(needs_layout_passes=False),
        scratch_types=[
            pltpu.VMEM((_RSUB,), jnp.int32),
            pltpu.VMEM((_RSUB, _C), jnp.float32),
            pltpu.VMEM((_RSUB, _C), jnp.float32),
            pltpu.VMEM((_C, _RSUB), jnp.float32),
            pltpu.VMEM((_C, _RSUB), jnp.float32),
            pltpu.SemaphoreType.DMA,
            pltpu.SemaphoreType.DMA,
            pltpu.SemaphoreType.DMA,
            pltpu.SemaphoreType.DMA,
        ],
    )
    out = run(xv, tab)
    return out.reshape(B, C, H * H)


# speedup vs baseline: 1.2708x; 1.0781x over previous
import functools

import jax
import jax.numpy as jnp
import numpy as np
from jax import lax
from jax.experimental import pallas as pl
from jax.experimental.pallas import tpu as pltpu
from jax.experimental.pallas import tpu_sc as plsc

_H = 64
_L = _H * _H
_B = 16
_C = 96
_NW = 32
_NSUB_TOTAL = 512
_SUBS_PER_W = _NSUB_TOTAL // _NW
_RSUB = 128


def _row_table() -> np.ndarray:
    q = np.arange(_RSUB)
    il = ((q >> 1) & 1) | (((q >> 3) & 1) << 1) | (((q >> 5) & 1) << 2)
    jl = (q & 1) | (((q >> 2) & 1) << 1) | (((q >> 4) & 1) << 2) | (((q >> 6) & 1) << 3)
    return (il * 16 + jl).astype(np.int32)


def _body(x_hbm, tab_hbm, out_hbm, tab_v, in_a, in_b, out_a, out_b,
          in_sem_a, in_sem_b, out_sem_a, out_sem_b):
    nc = 2
    wid = lax.axis_index("s") * nc + lax.axis_index("c")
    pltpu.sync_copy(tab_hbm, tab_v)
    sid0 = wid * _SUBS_PER_W

    def start_in(sid, in_ref, sem):
        cid = sid >> 2
        k = sid & 3
        base = cid * 512 + (k & 1) * 16 + ((k >> 1) & 1) * 32
        for il in range(8):
            pltpu.make_async_copy(
                x_hbm.at[pl.ds(pl.multiple_of(base + il * 64, 16), 16)],
                in_ref.at[pl.ds(il * 16, 16)],
                sem,
            ).start()

    def wait_in(in_ref, sem):
        for il in range(8):
            pltpu.make_async_copy(
                x_hbm.at[pl.ds(0, 16)],
                in_ref.at[pl.ds(il * 16, 16)],
                sem,
            ).wait()

    def out_slices(sid, out_ref):
        cid = sid >> 2
        k = sid & 3
        b = cid >> 3
        h = cid & 7
        colstart = (
            (h & 1) * 128 + (k & 1) * 256 + ((h >> 1) & 1) * 512
            + ((k >> 1) & 1) * 1024 + ((h >> 2) & 1) * 2048
        )
        dst = out_hbm.at[
            pl.ds(pl.multiple_of(b * _C, _C), _C),
            pl.ds(pl.multiple_of(colstart, 128), 128),
        ]
        return out_ref, dst

    def start_out(sid, out_ref, sem):
        src, dst = out_slices(sid, out_ref)
        pltpu.make_async_copy(src, dst, sem).start()

    def wait_out(out_ref, sem):
        pltpu.make_async_copy(
            out_ref, out_hbm.at[pl.ds(0, _C), pl.ds(0, 128)], sem
        ).wait()

    iota = jax.lax.iota(jnp.int32, 16)

    def compute(in_ref, out_ref):
        for g in range(8):
            rows_g = tab_v[pl.ds(g * 16, 16)]
            qvec = g * 16 + iota

            @plsc.parallel_loop(0, _C // 16, step=1, unroll=2)
            def cb_body(cb, rows_g=rows_g, qvec=qvec, g=g):
                cbs = jnp.full((16,), cb * 16, dtype=jnp.int32)
                for d in range(16):
                    cvec = cbs | (iota ^ d)
                    vals = plsc.load_gather(in_ref, [rows_g, cvec])
                    plsc.store_scatter(out_ref, [cvec, qvec], vals)

    start_in(sid0, in_a, in_sem_a)
    start_in(sid0 + 1, in_b, in_sem_b)

    def pair_body(t, _):
        u = t * 2

        wait_in(in_a, in_sem_a)

        @pl.when(t > 0)
        def _():
            wait_out(out_a, out_sem_a)

        compute(in_a, out_a)
        start_out(sid0 + u, out_a, out_sem_a)

        @pl.when(t < 7)
        def _():
            start_in(sid0 + u + 2, in_a, in_sem_a)

        wait_in(in_b, in_sem_b)

        @pl.when(t > 0)
        def _():
            wait_out(out_b, out_sem_b)

        compute(in_b, out_b)
        start_out(sid0 + u + 1, out_b, out_sem_b)

        @pl.when(t < 7)
        def _():
            start_in(sid0 + u + 3, in_b, in_sem_b)

        return 0

    lax.fori_loop(0, _SUBS_PER_W // 2, pair_body, 0, unroll=False)
    wait_out(out_a, out_sem_a)
    wait_out(out_b, out_sem_b)


@functools.partial(jax.jit, static_argnames=())
def kernel(x):
    B, C, H, _ = x.shape
    xv = x.transpose(0, 2, 3, 1).reshape(B * H * H, C)
    tab = jnp.asarray(_row_table())
    run = pl.kernel(
        _body,
        out_type=jax.ShapeDtypeStruct((B * C, H * H), jnp.float32),
        mesh=plsc.VectorSubcoreMesh(core_axis_name="c", subcore_axis_name="s"),
        compiler_params=pltpu.CompilerParams(needs_layout_passes=False),
        scratch_types=[
            pltpu.VMEM((_RSUB,), jnp.int32),
            pltpu.VMEM((_RSUB, _C), jnp.float32),
            pltpu.VMEM((_RSUB, _C), jnp.float32),
            pltpu.VMEM((_C, _RSUB), jnp.float32),
            pltpu.VMEM((_C, _RSUB), jnp.float32),
            pltpu.SemaphoreType.DMA,
            pltpu.SemaphoreType.DMA,
            pltpu.SemaphoreType.DMA,
            pltpu.SemaphoreType.DMA,
        ],
    )
    out = run(xv, tab)
    return out.reshape(B, C, H * H)


# speedup vs baseline: 1.7389x; 1.3684x over previous
import functools

import jax
import jax.numpy as jnp
import numpy as np
from jax import lax
from jax.experimental import pallas as pl
from jax.experimental.pallas import tpu as pltpu
from jax.experimental.pallas import tpu_sc as plsc

_H = 64
_L = _H * _H
_B = 16
_C = 96
_NW = 32
_NSUB_TOTAL = 512
_SUBS_PER_W = _NSUB_TOTAL // _NW
_RSUB = 128


def _row_table() -> np.ndarray:
    q = np.arange(_RSUB)
    il = ((q >> 1) & 1) | (((q >> 3) & 1) << 1) | (((q >> 5) & 1) << 2)
    jl = (q & 1) | (((q >> 2) & 1) << 1) | (((q >> 4) & 1) << 2) | (((q >> 6) & 1) << 3)
    return (il * 16 + jl).astype(np.int32)


def _body(x_hbm, tab_hbm, out_hbm, tab_v, in_a, in_b, out_a, out_b,
          in_sem_a, in_sem_b, out_sem_a, out_sem_b):
    nc = 2
    wid = lax.axis_index("s") * nc + lax.axis_index("c")
    pltpu.sync_copy(tab_hbm, tab_v)
    sid0 = wid * _SUBS_PER_W

    def start_in(sid, in_ref, sem):
        cid = sid >> 2
        k = sid & 3
        base = cid * 512 + (k & 1) * 16 + ((k >> 1) & 1) * 32
        for il in range(8):
            pltpu.make_async_copy(
                x_hbm.at[pl.ds(pl.multiple_of(base + il * 64, 16), 16)],
                in_ref.at[pl.ds(il * 16, 16)],
                sem,
            ).start()

    def wait_in(in_ref, sem):
        for il in range(8):
            pltpu.make_async_copy(
                x_hbm.at[pl.ds(0, 16)],
                in_ref.at[pl.ds(il * 16, 16)],
                sem,
            ).wait()

    def out_slices(sid, out_ref):
        cid = sid >> 2
        k = sid & 3
        b = cid >> 3
        h = cid & 7
        colstart = (
            (h & 1) * 128 + (k & 1) * 256 + ((h >> 1) & 1) * 512
            + ((k >> 1) & 1) * 1024 + ((h >> 2) & 1) * 2048
        )
        dst = out_hbm.at[
            pl.ds(pl.multiple_of(b * _C, _C), _C),
            pl.ds(pl.multiple_of(colstart, 128), 128),
        ]
        return out_ref, dst

    def start_out(sid, out_ref, sem):
        src, dst = out_slices(sid, out_ref)
        pltpu.make_async_copy(src, dst, sem).start()

    def wait_out(out_ref, sem):
        pltpu.make_async_copy(
            out_ref, out_hbm.at[pl.ds(0, _C), pl.ds(0, 128)], sem
        ).wait()

    iota = jax.lax.iota(jnp.int32, 16)
    rots = [(iota + d) & 15 for d in range(16)]

    def compute(in_ref, out_ref):
        for g in range(8):
            rows_g = tab_v[pl.ds(g * 16, 16)]
            qvec = g * 16 + iota

            @plsc.parallel_loop(0, _C // 16, step=1)
            def cb_body(cb, rows_g=rows_g, qvec=qvec, g=g):
                cbs = jnp.full((16,), cb * 16, dtype=jnp.int32)
                for d in range(16):
                    cvec = cbs + rots[d]
                    vals = plsc.load_gather(in_ref, [rows_g, cvec])
                    plsc.store_scatter(out_ref, [cvec, qvec], vals)

    start_in(sid0, in_a, in_sem_a)
    start_in(sid0 + 1, in_b, in_sem_b)

    def pair_body(t, _):
        u = t * 2

        wait_in(in_a, in_sem_a)

        @pl.when(t > 0)
        def _():
            wait_out(out_a, out_sem_a)

        compute(in_a, out_a)
        start_out(sid0 + u, out_a, out_sem_a)

        @pl.when(t < 7)
        def _():
            start_in(sid0 + u + 2, in_a, in_sem_a)

        wait_in(in_b, in_sem_b)

        @pl.when(t > 0)
        def _():
            wait_out(out_b, out_sem_b)

        compute(in_b, out_b)
        start_out(sid0 + u + 1, out_b, out_sem_b)

        @pl.when(t < 7)
        def _():
            start_in(sid0 + u + 3, in_b, in_sem_b)

        return 0

    lax.fori_loop(0, _SUBS_PER_W // 2, pair_body, 0, unroll=False)
    wait_out(out_a, out_sem_a)
    wait_out(out_b, out_sem_b)


@functools.partial(jax.jit, static_argnames=())
def kernel(x):
    B, C, H, _ = x.shape
    xv = x.transpose(0, 2, 3, 1).reshape(B * H * H, C)
    tab = jnp.asarray(_row_table())
    run = pl.kernel(
        _body,
        out_type=jax.ShapeDtypeStruct((B * C, H * H), jnp.float32),
        mesh=plsc.VectorSubcoreMesh(core_axis_name="c", subcore_axis_name="s"),
        compiler_params=pltpu.CompilerParams(needs_layout_passes=False),
        scratch_types=[
            pltpu.VMEM((_RSUB,), jnp.int32),
            pltpu.VMEM((_RSUB, _C), jnp.float32),
            pltpu.VMEM((_RSUB, _C), jnp.float32),
            pltpu.VMEM((_C, _RSUB), jnp.float32),
            pltpu.VMEM((_C, _RSUB), jnp.float32),
            pltpu.SemaphoreType.DMA,
            pltpu.SemaphoreType.DMA,
            pltpu.SemaphoreType.DMA,
            pltpu.SemaphoreType.DMA,
        ],
    )
    out = run(xv, tab)
    return out.reshape(B, C, H * H)
